# baseline (speedup 1.0000x reference)
import jax
import jax.numpy as jnp
from jax.experimental import pallas as pl

_HIDDEN = 768
_RANK = _HIDDEN // 4
_THRESH = 0.5
_N = 4 * 8192
_B = 2048
_NB = _N // _B


def _gate_kernel(x_ref, wt_ref, mask_ref, score_ref):
    x = x_ref[...].astype(jnp.bfloat16)
    h = jnp.dot(x, wt_ref[...], preferred_element_type=jnp.float32)
    s2 = jnp.sum(h * h, axis=-1)
    score = jnp.sqrt(s2)
    m = score >= _THRESH
    mask_ref[0, 0, :] = m
    score_ref[0, 0, :] = jnp.where(m, score, -jnp.inf)


def kernel(x, W):
    xf = x.reshape(_N, _HIDDEN)
    wt = W.T.astype(jnp.bfloat16)
    mask, score = pl.pallas_call(
        _gate_kernel,
        grid=(_NB,),
        in_specs=[
            pl.BlockSpec((_B, _HIDDEN), lambda i: (i, 0)),
            pl.BlockSpec((_HIDDEN, _RANK), lambda i: (0, 0)),
        ],
        out_specs=[
            pl.BlockSpec((1, 1, _B), lambda i: (i, 0, 0)),
            pl.BlockSpec((1, 1, _B), lambda i: (i, 0, 0)),
        ],
        out_shape=[
            jax.ShapeDtypeStruct((_NB, 1, _B), jnp.bool_),
            jax.ShapeDtypeStruct((_NB, 1, _B), jnp.float32),
        ],
    )(xf, wt)
    return mask.reshape(x.shape[:-1]), score.reshape(x.shape[:-1])


# speedup vs baseline: 1.1613x; 1.1613x over previous
import jax
import jax.numpy as jnp
from jax.experimental import pallas as pl

_HIDDEN = 768
_RANK = _HIDDEN // 4
_THRESH = 0.5
_N = 4 * 8192
_B = 2048
_NB = _N // _B


def _gate_kernel(x_ref, wt_ref, mask_ref, score_ref):
    x = x_ref[...].astype(jnp.bfloat16)
    h = jnp.dot(x, wt_ref[...], preferred_element_type=jnp.float32)
    s2 = jnp.sum(h * h, axis=-1, keepdims=True)
    score = jnp.sqrt(s2)
    m = score >= _THRESH
    mask_ref[...] = m.astype(jnp.float32)
    score_ref[...] = jnp.where(m, score, -jnp.inf)


def kernel(x, W):
    xf = x.reshape(_N, _HIDDEN)
    wt = W.T.astype(jnp.bfloat16)
    mask_f, score = pl.pallas_call(
        _gate_kernel,
        grid=(_NB,),
        in_specs=[
            pl.BlockSpec((_B, _HIDDEN), lambda i: (i, 0)),
            pl.BlockSpec((_HIDDEN, _RANK), lambda i: (0, 0)),
        ],
        out_specs=[
            pl.BlockSpec((_B, 1), lambda i: (i, 0)),
            pl.BlockSpec((_B, 1), lambda i: (i, 0)),
        ],
        out_shape=[
            jax.ShapeDtypeStruct((_N, 1), jnp.float32),
            jax.ShapeDtypeStruct((_N, 1), jnp.float32),
        ],
    )(xf, wt)
    lead = x.shape[:-1]
    return mask_f.reshape(lead).astype(jnp.bool_), score.reshape(lead)


# speedup vs baseline: 1.1733x; 1.0104x over previous
import jax
import jax.numpy as jnp
from jax.experimental import pallas as pl

_HIDDEN = 768
_RANK = _HIDDEN // 4
_THRESH = 0.5
_N = 4 * 8192
_B = 2048
_NB = _N // _B


def _gate_kernel(x_ref, wt_ref, mask_ref, score_ref):
    x = x_ref[...]
    h = jnp.dot(x, wt_ref[...], preferred_element_type=jnp.float32)
    s2 = jnp.sum(h * h, axis=-1, keepdims=True)
    score = jnp.sqrt(s2)
    m = score >= _THRESH
    mask_ref[...] = m.astype(jnp.float32)
    score_ref[...] = jnp.where(m, score, -jnp.inf)


def kernel(x, W):
    xf = x.reshape(_N, _HIDDEN)
    wt = W.T
    mask_f, score = pl.pallas_call(
        _gate_kernel,
        grid=(_NB,),
        in_specs=[
            pl.BlockSpec((_B, _HIDDEN), lambda i: (i, 0)),
            pl.BlockSpec((_HIDDEN, _RANK), lambda i: (0, 0)),
        ],
        out_specs=[
            pl.BlockSpec((_B, 1), lambda i: (i, 0)),
            pl.BlockSpec((_B, 1), lambda i: (i, 0)),
        ],
        out_shape=[
            jax.ShapeDtypeStruct((_N, 1), jnp.float32),
            jax.ShapeDtypeStruct((_N, 1), jnp.float32),
        ],
    )(xf, wt)
    lead = x.shape[:-1]
    return mask_f.reshape(lead).astype(jnp.bool_), score.reshape(lead)
